# baseline (speedup 1.0000x reference)
import jax
import jax.numpy as jnp
from jax.experimental import pallas as pl
from jax.experimental.pallas import tpu as pltpu

N = 10000
E = 320000
D_H = 256
ITER = 10
SKIP = 2
G = 16


def _pairnorm_elu_body(x_ref, o_ref):
    x = x_ref[...]
    x = x - jnp.mean(x, axis=0, keepdims=True)
    x = x / jnp.sqrt(1e-5 + jnp.mean(jnp.sum(x * x, axis=-1)))
    o_ref[...] = jnp.where(x > 0, x, jnp.exp(jnp.minimum(x, 0.0)) - 1.0)


def _pairnorm_elu(x):
    return pl.pallas_call(
        _pairnorm_elu_body,
        out_shape=jax.ShapeDtypeStruct(x.shape, x.dtype),
    )(x)


def _gine(x, src, dst, edge_attr, We, be, W, b, n):
    e = edge_attr @ We + be
    msg = jax.nn.relu(x[src] + e)
    agg = jax.ops.segment_sum(msg, dst, num_segments=n)
    return (x + agg) @ W + b


def kernel(x, edge_index, edge_attr, batch, We0, be0, W0, b0, WeH, beH, WH, bH,
           We_mus, be_mus, W_mus, b_mus, We_sat, be_sat, W_sat, b_sat, W_lin, b_lin):
    src, dst = edge_index[0], edge_index[1]
    h = _gine(x, src, dst, edge_attr, We0, be0, W0, b0, N)
    h = _pairnorm_elu(h)
    h_old = h
    for i in range(ITER):
        h = _gine(h, src, dst, edge_attr, WeH[i], beH[i], WH[i], bH[i], N)
        h = _pairnorm_elu(h)
        if (i + 1) % SKIP == 0:
            h = h + h_old
    mus = _gine(h, src, dst, edge_attr, We_mus, be_mus, W_mus, b_mus, N).squeeze(-1)
    sat_nodes = _gine(h, src, dst, edge_attr, We_sat, be_sat, W_sat, b_sat, N)
    sums = jax.ops.segment_sum(sat_nodes, batch, num_segments=G)
    counts = jax.ops.segment_sum(jnp.ones((N,), dtype=jnp.float32), batch, num_segments=G)
    pooled = sums / jnp.clip(counts, 1.0)[:, None]
    sat = (pooled @ W_lin + b_lin).squeeze(-1)
    return (mus, sat)


# speedup vs baseline: 2.0001x; 2.0001x over previous
import functools

import jax
import jax.numpy as jnp
from jax import lax
from jax.experimental import pallas as pl
from jax.experimental.pallas import tpu as pltpu
from jax.experimental.pallas import tpu_sc as plsc

N = 10000
E = 320000
D_IN = 128
D_H = 256
ITER = 10
SKIP = 2
G = 16

CB = 128
NCH = E // CB
NT = 16
QFULL = NCH // NT
QREM = NCH % NT
WCH = 80
NWCH = N // WCH
WFULL = NWCH // NT
WREM = NWCH % NT


def _edge_body(h2, e2, src1, dst1, out, src_v, dst_v, rows_v, e_v, acc_sh, sem):
    c = lax.axis_index("c")
    s = lax.axis_index("s")
    coff = c * N

    zero16 = jnp.zeros((16,), jnp.float32)

    def zrow(j, carry):
        for k in range(8):
            rows_v[j, pl.ds(k * 16, 16)] = zero16
        return carry

    lax.fori_loop(0, WCH, zrow, 0)
    nw = WFULL + jnp.where(s < WREM, 1, 0)

    def wbase(w):
        return pl.multiple_of((s + NT * w) * WCH, 8)

    def zchunk(w, carry):
        pltpu.sync_copy(rows_v.at[pl.ds(0, WCH)], acc_sh.at[pl.ds(wbase(w), WCH)])
        return carry

    lax.fori_loop(0, nw, zchunk, 0)
    plsc.subcore_barrier()

    nq = QFULL + jnp.where(s < QREM, 1, 0)

    def chunk(k, carry):
        q = s + NT * k
        base = pl.multiple_of(q * CB, 8)
        pltpu.sync_copy(src1.at[pl.ds(base, CB)], src_v)
        pltpu.sync_copy(dst1.at[pl.ds(base, CB)], dst_v)
        for k2 in range(8):
            sl = pl.ds(k2 * 16, 16)
            src_v[sl] = src_v[sl] + coff
        pltpu.sync_copy(e2.at[pl.ds(pl.multiple_of(c * E + base, 8), CB)], e_v)
        pltpu.async_copy(h2.at[src_v], rows_v, sem).wait()

        def frow(j, cc):
            for k2 in range(8):
                sl = pl.ds(k2 * 16, 16)
                rows_v[j, sl] = jnp.maximum(rows_v[j, sl] + e_v[j, sl], 0.0)
            return cc

        lax.fori_loop(0, CB, frow, 0)
        pltpu.sync_copy(rows_v, acc_sh.at[dst_v], add=True)
        return carry

    lax.fori_loop(0, nq, chunk, 0)
    plsc.subcore_barrier()

    def wchunk(w, carry):
        base = wbase(w)
        pltpu.sync_copy(acc_sh.at[pl.ds(base, WCH)], rows_v.at[pl.ds(0, WCH)])
        pltpu.sync_copy(rows_v.at[pl.ds(0, WCH)],
                        out.at[pl.ds(pl.multiple_of(coff + base, 8), WCH)])
        return carry

    lax.fori_loop(0, nw, wchunk, 0)


_edge_phase = functools.partial(
    pl.kernel,
    _edge_body,
    out_type=jax.ShapeDtypeStruct((2 * N, 128), jnp.float32),
    mesh=plsc.VectorSubcoreMesh(core_axis_name="c", subcore_axis_name="s"),
    scratch_types=[
        pltpu.VMEM((CB,), jnp.int32),
        pltpu.VMEM((CB,), jnp.int32),
        pltpu.VMEM((CB, 128), jnp.float32),
        pltpu.VMEM((CB, 128), jnp.float32),
        pltpu.VMEM_SHARED((N, 128), jnp.float32),
        pltpu.SemaphoreType.DMA,
    ],
)()


_BE = 4000


def _e_matmul_body(a_ref, w_ref, b_ref, o_ref):
    y = jnp.dot(a_ref[...], w_ref[...], preferred_element_type=jnp.float32)
    y = y + b_ref[...]
    if o_ref.shape[0] == 2 and y.shape[1] == 256:
        o_ref[0] = y[:, :128]
        o_ref[1] = y[:, 128:]
    else:
        o_ref[0] = y
        o_ref[1] = y


def _e_matmul(edge_attr, We, be):
    do = We.shape[1]
    out = pl.pallas_call(
        _e_matmul_body,
        grid=(E // _BE,),
        in_specs=[
            pl.BlockSpec((_BE, 16), lambda i: (i, 0)),
            pl.BlockSpec((16, do), lambda i: (0, 0)),
            pl.BlockSpec((1, do), lambda i: (0, 0)),
        ],
        out_specs=pl.BlockSpec((2, _BE, 128), lambda i: (0, i, 0)),
        out_shape=jax.ShapeDtypeStruct((2, E, 128), jnp.float32),
    )(edge_attr, We, be.reshape(1, do))
    return out.reshape(2 * E, 128)


def _elu(y):
    return jnp.where(y > 0, y, jnp.exp(jnp.minimum(y, 0.0)) - 1.0)


def _pairnorm_elu(y):
    y = y - jnp.mean(y, axis=0, keepdims=True)
    y = y / jnp.sqrt(1e-5 + jnp.mean(jnp.sum(y * y, axis=-1)))
    return _elu(y)


def _node0_body(x_ref, agg_ref, w_ref, b_ref, o_ref):
    xa = x_ref[...] + agg_ref[...]
    y = jnp.dot(xa, w_ref[...], preferred_element_type=jnp.float32) + b_ref[...]
    z = _pairnorm_elu(y)
    o_ref[pl.ds(0, N), :] = z[:, :128]
    o_ref[pl.ds(N, N), :] = z[:, 128:]


def _node0(x, agg2, W, b):
    return pl.pallas_call(
        _node0_body,
        out_shape=jax.ShapeDtypeStruct((2 * N, 128), jnp.float32),
    )(x, agg2[:N], W, b.reshape(1, D_H))


def _nodeh_body(h_ref, agg_ref, w_ref, b_ref, o_ref):
    x0 = h_ref[pl.ds(0, N), :] + agg_ref[pl.ds(0, N), :]
    x1 = h_ref[pl.ds(N, N), :] + agg_ref[pl.ds(N, N), :]
    y = jnp.dot(x0, w_ref[pl.ds(0, 128), :], preferred_element_type=jnp.float32)
    y = y + jnp.dot(x1, w_ref[pl.ds(128, 128), :], preferred_element_type=jnp.float32)
    y = y + b_ref[...]
    z = _pairnorm_elu(y)
    o_ref[pl.ds(0, N), :] = z[:, :128]
    o_ref[pl.ds(N, N), :] = z[:, 128:]


def _nodeh(h2, agg2, W, b):
    return pl.pallas_call(
        _nodeh_body,
        out_shape=jax.ShapeDtypeStruct((2 * N, 128), jnp.float32),
    )(h2, agg2, W, b.reshape(1, D_H))


def _mus_body(h_ref, am_ref, wm_ref, bm_ref, mus_ref):
    xm0 = h_ref[pl.ds(0, N), :] + am_ref[pl.ds(0, N), :]
    xm1 = h_ref[pl.ds(N, N), :] + am_ref[pl.ds(N, N), :]
    mus = jnp.dot(xm0, wm_ref[pl.ds(0, 128), :], preferred_element_type=jnp.float32)
    mus = mus + jnp.dot(xm1, wm_ref[pl.ds(128, 128), :], preferred_element_type=jnp.float32)
    mus_ref[...] = mus + bm_ref[0, 0]


def _sat_body(h_ref, as_ref, ws_ref, bs_ref, wl_ref, bl_ref, batch_ref, sat_ref):
    xs0 = h_ref[pl.ds(0, N), :] + as_ref[pl.ds(0, N), :]
    xs1 = h_ref[pl.ds(N, N), :] + as_ref[pl.ds(N, N), :]
    sn = jnp.dot(xs0, ws_ref[pl.ds(0, 128), :], preferred_element_type=jnp.float32)
    sn = sn + jnp.dot(xs1, ws_ref[pl.ds(128, 128), :], preferred_element_type=jnp.float32)
    sn = sn + bs_ref[...]

    b = batch_ref[...]
    onehot = (b == lax.broadcasted_iota(jnp.int32, (N, G), 1)).astype(jnp.float32)
    sums = jnp.dot(onehot.T, sn, preferred_element_type=jnp.float32)
    counts = jnp.sum(onehot, axis=0)
    pooled = sums / jnp.maximum(counts, 1.0)[:, None]
    sat = jnp.dot(pooled, wl_ref[...], preferred_element_type=jnp.float32)
    sat_ref[...] = sat + bl_ref[0, 0]


def _final(h2, aggm2, aggs2, W_mus, b_mus, W_sat, b_sat, W_lin, b_lin, batch):
    mus = pl.pallas_call(
        _mus_body,
        out_shape=jax.ShapeDtypeStruct((N, 1), jnp.float32),
    )(h2, aggm2, W_mus, b_mus.reshape(1, 1))
    sat = pl.pallas_call(
        _sat_body,
        out_shape=jax.ShapeDtypeStruct((G, 1), jnp.float32),
    )(h2, aggs2, W_sat, b_sat.reshape(1, G), W_lin, b_lin.reshape(1, 1),
      batch.reshape(N, 1))
    return mus.reshape(N), sat.reshape(G)


def kernel(x, edge_index, edge_attr, batch, We0, be0, W0, b0, WeH, beH, WH, bH,
           We_mus, be_mus, W_mus, b_mus, We_sat, be_sat, W_sat, b_sat, W_lin, b_lin):
    src2 = edge_index[0]
    dst2 = edge_index[1]

    x2 = jnp.concatenate([x, x], axis=0)
    e2 = _e_matmul(edge_attr, We0, be0)
    agg2 = _edge_phase(x2, e2, src2, dst2)
    h2 = _node0(x, agg2, W0, b0)
    h_old2 = h2

    for i in range(ITER):
        e2 = _e_matmul(edge_attr, WeH[i], beH[i])
        agg2 = _edge_phase(h2, e2, src2, dst2)
        h2 = _nodeh(h2, agg2, WH[i], bH[i])
        if (i + 1) % SKIP == 0:
            h2 = h2 + h_old2

    e2 = _e_matmul(edge_attr, We_mus, be_mus)
    aggm2 = _edge_phase(h2, e2, src2, dst2)
    e2 = _e_matmul(edge_attr, We_sat, be_sat)
    aggs2 = _edge_phase(h2, e2, src2, dst2)
    return _final(h2, aggm2, aggs2, W_mus, b_mus, W_sat, b_sat, W_lin, b_lin, batch)


# speedup vs baseline: 3.2363x; 1.6181x over previous
import functools

import jax
import jax.numpy as jnp
from jax import lax
from jax.experimental import pallas as pl
from jax.experimental.pallas import tpu as pltpu
from jax.experimental.pallas import tpu_sc as plsc

N = 10000
E = 320000
D_IN = 128
D_H = 256
ITER = 10
SKIP = 2
G = 16

NT = 16
CB = 48
QPT = 420
EPAD = NT * CB * QPT
EXTRA = EPAD - E
NQUAD = QPT // 4

WCH = 80
NWCH = N // WCH
WFULL = NWCH // NT
WREM = NWCH % NT


def _edge_body(h2, e2, src1, dst1, out,
               src_v, dst_v, rows_v, e_v, sbuf, acc_sh,
               i0, i1, i2, i3, g0, g1, s0, s1):
    c = lax.axis_index("c")
    s = lax.axis_index("s")
    coff = c * N
    ebase0 = s * QPT * CB

    isems = (i0, i1, i2, i3)
    gsems = (g0, g1)
    ssems = (s0, s1)

    zero16 = jnp.zeros((16,), jnp.float32)

    def zrow(j, carry):
        for k in range(8):
            sbuf[0, j, pl.ds(k * 16, 16)] = zero16
        return carry

    lax.fori_loop(0, CB, zrow, 0)

    nw = WFULL + jnp.where(s < WREM, 1, 0)

    def wbase(w):
        return pl.multiple_of((s + NT * w) * WCH, 8)

    def zchunk(w, carry):
        for r in range(2):
            pltpu.sync_copy(sbuf.at[0, pl.ds(0, 40)],
                            acc_sh.at[pl.ds(wbase(w) + 40 * r, 40)])
        return carry

    lax.fori_loop(0, nw, zchunk, 0)
    plsc.subcore_barrier()

    def idx_start(k, islot):
        base = pl.multiple_of(ebase0 + k * CB, 8)
        pltpu.async_copy(src1.at[pl.ds(base, CB)], src_v.at[islot], isems[islot])
        pltpu.async_copy(dst1.at[pl.ds(base, CB)], dst_v.at[islot], isems[islot])

    def idx_wait(k, islot):
        base = pl.multiple_of(ebase0 + k * CB, 8)
        pltpu.make_async_copy(src1.at[pl.ds(base, CB)], src_v.at[islot],
                              isems[islot]).wait()
        pltpu.make_async_copy(dst1.at[pl.ds(base, CB)], dst_v.at[islot],
                              isems[islot]).wait()

    def coff_add(islot):
        for k2 in range(CB // 16):
            sl = pl.ds(k2 * 16, 16)
            src_v[islot, sl] = src_v[islot, sl] + coff

    def g_start(k, gslot, islot):
        sem = gsems[gslot]
        pltpu.async_copy(h2.at[src_v.at[islot]], rows_v.at[gslot], sem)
        ebase = pl.multiple_of(c * EPAD + ebase0 + k * CB, 8)
        pltpu.async_copy(e2.at[pl.ds(ebase, CB)], e_v.at[gslot], sem)

    def g_wait(k, gslot, islot):
        sem = gsems[gslot]
        pltpu.make_async_copy(h2.at[src_v.at[islot]], rows_v.at[gslot], sem).wait()
        ebase = pl.multiple_of(c * EPAD + ebase0 + k * CB, 8)
        pltpu.make_async_copy(e2.at[pl.ds(ebase, CB)], e_v.at[gslot], sem).wait()

    def compute(gslot):
        def frow(j2, cc):
            for u2 in range(2):
                j = 2 * j2 + u2
                for k2 in range(8):
                    sl = pl.ds(k2 * 16, 16)
                    sbuf[gslot, j, sl] = jnp.maximum(
                        rows_v[gslot, j, sl] + e_v[gslot, j, sl], 0.0)
            return cc

        lax.fori_loop(0, CB // 2, frow, 0)

    def sc_start(gslot, islot):
        pltpu.async_copy(sbuf.at[gslot], acc_sh.at[dst_v.at[islot]],
                         ssems[gslot], add=True)

    def sc_wait(gslot, islot):
        pltpu.make_async_copy(sbuf.at[gslot], acc_sh.at[dst_v.at[islot]],
                              ssems[gslot]).wait()

    idx_start(0, 0)
    idx_start(1, 1)
    idx_wait(0, 0)
    coff_add(0)
    g_start(0, 0, 0)

    def quad(u, carry):
        for j in range(4):
            k = 4 * u + j
            gs = j & 1
            if j < 2:
                @pl.when(u > 0)
                def _():
                    sc_wait(gs, j)
            else:
                sc_wait(gs, j)
            if j < 2:
                idx_start(k + 2, (j + 2) % 4)
            else:
                @pl.when(u < NQUAD - 1)
                def _():
                    idx_start(k + 2, (j + 2) % 4)
            if j == 3:
                @pl.when(u < NQUAD - 1)
                def _():
                    idx_wait(k + 1, 0)
                    coff_add(0)
                    g_start(k + 1, 0, 0)
            else:
                idx_wait(k + 1, (j + 1) % 4)
                coff_add((j + 1) % 4)
                g_start(k + 1, (j + 1) & 1, (j + 1) % 4)
            g_wait(k, gs, j)
            compute(gs)
            sc_start(gs, j)
        return carry

    lax.fori_loop(0, NQUAD, quad, 0)
    sc_wait(0, 2)
    sc_wait(1, 3)
    plsc.subcore_barrier()

    def wchunk(w, carry):
        base = wbase(w)
        for r in range(2):
            pltpu.sync_copy(acc_sh.at[pl.ds(base + 40 * r, 40)],
                            sbuf.at[r, pl.ds(0, 40)])
            pltpu.sync_copy(sbuf.at[r, pl.ds(0, 40)],
                            out.at[pl.ds(pl.multiple_of(coff + base + 40 * r, 8), 40)])
        return carry

    lax.fori_loop(0, nw, wchunk, 0)


_edge_phase = pl.kernel(
    _edge_body,
    out_type=jax.ShapeDtypeStruct((2 * N, 128), jnp.float32),
    mesh=plsc.VectorSubcoreMesh(core_axis_name="c", subcore_axis_name="s"),
    scratch_types=[
        pltpu.VMEM((4, CB), jnp.int32),
        pltpu.VMEM((4, CB), jnp.int32),
        pltpu.VMEM((2, CB, 128), jnp.float32),
        pltpu.VMEM((2, CB, 128), jnp.float32),
        pltpu.VMEM((2, CB, 128), jnp.float32),
        pltpu.VMEM_SHARED((N, 128), jnp.float32),
        pltpu.SemaphoreType.DMA,
        pltpu.SemaphoreType.DMA,
        pltpu.SemaphoreType.DMA,
        pltpu.SemaphoreType.DMA,
        pltpu.SemaphoreType.DMA,
        pltpu.SemaphoreType.DMA,
        pltpu.SemaphoreType.DMA,
        pltpu.SemaphoreType.DMA,
    ],
)


_BE = 3840


def _e_matmul_body(a_ref, w_ref, b_ref, o_ref):
    i = pl.program_id(0)
    y = jnp.dot(a_ref[...], w_ref[...], preferred_element_type=jnp.float32)
    y = y + b_ref[...]
    rows = i * _BE + lax.broadcasted_iota(jnp.int32, y.shape, 0)
    y = jnp.where(rows < E, y, -3e38)
    if y.shape[1] == 256:
        o_ref[0] = y[:, :128]
        o_ref[1] = y[:, 128:]
    else:
        o_ref[0] = y
        o_ref[1] = y


def _e_matmul(ea_pad, We, be):
    do = We.shape[1]
    out = pl.pallas_call(
        _e_matmul_body,
        grid=(EPAD // _BE,),
        in_specs=[
            pl.BlockSpec((_BE, 16), lambda i: (i, 0)),
            pl.BlockSpec((16, do), lambda i: (0, 0)),
            pl.BlockSpec((1, do), lambda i: (0, 0)),
        ],
        out_specs=pl.BlockSpec((2, _BE, 128), lambda i: (0, i, 0)),
        out_shape=jax.ShapeDtypeStruct((2, EPAD, 128), jnp.float32),
    )(ea_pad, We, be.reshape(1, do))
    return out.reshape(2 * EPAD, 128)


def _elu(y):
    return jnp.where(y > 0, y, jnp.exp(jnp.minimum(y, 0.0)) - 1.0)


def _pairnorm_elu(y):
    y = y - jnp.mean(y, axis=0, keepdims=True)
    y = y / jnp.sqrt(1e-5 + jnp.mean(jnp.sum(y * y, axis=-1)))
    return _elu(y)


def _node0_body(x_ref, agg_ref, w_ref, b_ref, o_ref):
    xa = x_ref[...] + agg_ref[...]
    y = jnp.dot(xa, w_ref[...], preferred_element_type=jnp.float32) + b_ref[...]
    z = _pairnorm_elu(y)
    o_ref[pl.ds(0, N), :] = z[:, :128]
    o_ref[pl.ds(N, N), :] = z[:, 128:]


def _node0(x, agg2, W, b):
    return pl.pallas_call(
        _node0_body,
        out_shape=jax.ShapeDtypeStruct((2 * N, 128), jnp.float32),
    )(x, agg2[:N], W, b.reshape(1, D_H))


def _nodeh_body(h_ref, agg_ref, w_ref, b_ref, o_ref):
    x0 = h_ref[pl.ds(0, N), :] + agg_ref[pl.ds(0, N), :]
    x1 = h_ref[pl.ds(N, N), :] + agg_ref[pl.ds(N, N), :]
    y = jnp.dot(x0, w_ref[pl.ds(0, 128), :], preferred_element_type=jnp.float32)
    y = y + jnp.dot(x1, w_ref[pl.ds(128, 128), :], preferred_element_type=jnp.float32)
    y = y + b_ref[...]
    z = _pairnorm_elu(y)
    o_ref[pl.ds(0, N), :] = z[:, :128]
    o_ref[pl.ds(N, N), :] = z[:, 128:]


def _nodeh(h2, agg2, W, b):
    return pl.pallas_call(
        _nodeh_body,
        out_shape=jax.ShapeDtypeStruct((2 * N, 128), jnp.float32),
    )(h2, agg2, W, b.reshape(1, D_H))


def _mus_body(h_ref, am_ref, wm_ref, bm_ref, mus_ref):
    xm0 = h_ref[pl.ds(0, N), :] + am_ref[pl.ds(0, N), :]
    xm1 = h_ref[pl.ds(N, N), :] + am_ref[pl.ds(N, N), :]
    mus = jnp.dot(xm0, wm_ref[pl.ds(0, 128), :], preferred_element_type=jnp.float32)
    mus = mus + jnp.dot(xm1, wm_ref[pl.ds(128, 128), :], preferred_element_type=jnp.float32)
    mus_ref[...] = mus + bm_ref[0, 0]


def _sat_body(h_ref, as_ref, ws_ref, bs_ref, wl_ref, bl_ref, batch_ref, sat_ref):
    xs0 = h_ref[pl.ds(0, N), :] + as_ref[pl.ds(0, N), :]
    xs1 = h_ref[pl.ds(N, N), :] + as_ref[pl.ds(N, N), :]
    sn = jnp.dot(xs0, ws_ref[pl.ds(0, 128), :], preferred_element_type=jnp.float32)
    sn = sn + jnp.dot(xs1, ws_ref[pl.ds(128, 128), :], preferred_element_type=jnp.float32)
    sn = sn + bs_ref[...]

    b = batch_ref[...]
    onehot = (b == lax.broadcasted_iota(jnp.int32, (N, G), 1)).astype(jnp.float32)
    sums = jnp.dot(onehot.T, sn, preferred_element_type=jnp.float32)
    counts = jnp.sum(onehot, axis=0)
    pooled = sums / jnp.maximum(counts, 1.0)[:, None]
    sat = jnp.dot(pooled, wl_ref[...], preferred_element_type=jnp.float32)
    sat_ref[...] = sat + bl_ref[0, 0]


def _final(h2, aggm2, aggs2, W_mus, b_mus, W_sat, b_sat, W_lin, b_lin, batch):
    mus = pl.pallas_call(
        _mus_body,
        out_shape=jax.ShapeDtypeStruct((N, 1), jnp.float32),
    )(h2, aggm2, W_mus, b_mus.reshape(1, 1))
    sat = pl.pallas_call(
        _sat_body,
        out_shape=jax.ShapeDtypeStruct((G, 1), jnp.float32),
    )(h2, aggs2, W_sat, b_sat.reshape(1, G), W_lin, b_lin.reshape(1, 1),
      batch.reshape(N, 1))
    return mus.reshape(N), sat.reshape(G)


def kernel(x, edge_index, edge_attr, batch, We0, be0, W0, b0, WeH, beH, WH, bH,
           We_mus, be_mus, W_mus, b_mus, We_sat, be_sat, W_sat, b_sat, W_lin, b_lin):
    src1 = jnp.concatenate([edge_index[0], jnp.zeros((EXTRA,), jnp.int32)])
    dst1 = jnp.concatenate([edge_index[1], jnp.zeros((EXTRA,), jnp.int32)])
    ea_pad = jnp.concatenate([edge_attr, jnp.zeros((EXTRA, 16), jnp.float32)])

    x2 = jnp.concatenate([x, x], axis=0)
    e2 = _e_matmul(ea_pad, We0, be0)
    agg2 = _edge_phase(x2, e2, src1, dst1)
    h2 = _node0(x, agg2, W0, b0)
    h_old2 = h2

    for i in range(ITER):
        e2 = _e_matmul(ea_pad, WeH[i], beH[i])
        agg2 = _edge_phase(h2, e2, src1, dst1)
        h2 = _nodeh(h2, agg2, WH[i], bH[i])
        if (i + 1) % SKIP == 0:
            h2 = h2 + h_old2

    e2 = _e_matmul(ea_pad, We_mus, be_mus)
    aggm2 = _edge_phase(h2, e2, src1, dst1)
    e2 = _e_matmul(ea_pad, We_sat, be_sat)
    aggs2 = _edge_phase(h2, e2, src1, dst1)
    return _final(h2, aggm2, aggs2, W_mus, b_mus, W_sat, b_sat, W_lin, b_lin, batch)
